# initial kernel scaffold (unmeasured)
import jax
import jax.numpy as jnp
from jax import lax
from jax.experimental import pallas as pl
from jax.experimental.pallas import tpu as pltpu

N_DEV = 16


def kernel(x, w_mat, scale_x, scale_w):
    m, k_per = x.shape
    _, n = w_mat.shape
    n_hops = N_DEV - 1

    def body(x_ref, w_ref, sx_ref, sw_ref, out_ref,
             rx_x, rx_w, sx_sems, rxv_sems, sw_sems, rwv_sems):
        my = lax.axis_index("i")
        left = (my - 1) % N_DEV
        right = (my + 1) % N_DEV

        barrier = pltpu.get_barrier_semaphore()
        for nbr in (left, right):
            pl.semaphore_signal(barrier, inc=1, device_id=(nbr,),
                                device_id_type=pl.DeviceIdType.MESH)
        pl.semaphore_wait(barrier, 2)

        def dotf(a, b):
            return lax.dot_general(
                a, b, (((1,), (0,)), ((), ())),
                preferred_element_type=jnp.float32)

        for h in range(n_hops):
            src_x = x_ref if h == 0 else rx_x.at[h - 1]
            src_w = w_ref if h == 0 else rx_w.at[h - 1]
            rdx = pltpu.make_async_remote_copy(
                src_ref=src_x, dst_ref=rx_x.at[h],
                send_sem=sx_sems.at[h], recv_sem=rxv_sems.at[h],
                device_id=(right,), device_id_type=pl.DeviceIdType.MESH)
            rdw = pltpu.make_async_remote_copy(
                src_ref=src_w, dst_ref=rx_w.at[h],
                send_sem=sw_sems.at[h], recv_sem=rwv_sems.at[h],
                device_id=(right,), device_id_type=pl.DeviceIdType.MESH)
            rdx.start()
            rdw.start()
            if h == 0:
                out_ref[...] = dotf(x_ref[...], w_ref[...])
            else:
                out_ref[...] += dotf(rx_x[h - 1], rx_w[h - 1])
            rdx.wait()
            rdw.wait()

        out_ref[...] += dotf(rx_x[n_hops - 1], rx_w[n_hops - 1])
        scale = sx_ref[0] * sw_ref[0]
        out_ref[...] = jnp.maximum(out_ref[...] * scale, 0.0)

    return pl.pallas_call(
        body,
        out_shape=jax.ShapeDtypeStruct((m, n), jnp.float32),
        in_specs=[
            pl.BlockSpec(memory_space=pltpu.VMEM),
            pl.BlockSpec(memory_space=pltpu.VMEM),
            pl.BlockSpec(memory_space=pltpu.SMEM),
            pl.BlockSpec(memory_space=pltpu.SMEM),
        ],
        out_specs=pl.BlockSpec(memory_space=pltpu.VMEM),
        scratch_shapes=[
            pltpu.VMEM((n_hops, m, k_per), x.dtype),
            pltpu.VMEM((n_hops, k_per, n), w_mat.dtype),
            pltpu.SemaphoreType.DMA((n_hops,)),
            pltpu.SemaphoreType.DMA((n_hops,)),
            pltpu.SemaphoreType.DMA((n_hops,)),
            pltpu.SemaphoreType.DMA((n_hops,)),
        ],
        compiler_params=pltpu.CompilerParams(
            collective_id=0,
            vmem_limit_bytes=100 * 1024 * 1024,
        ),
    )(x, w_mat, scale_x, scale_w)


# baseline (device time: 330846 ns/iter reference)
import jax
import jax.numpy as jnp
from jax import lax
from jax.experimental import pallas as pl
from jax.experimental.pallas import tpu as pltpu

N_DEV = 16


def kernel(x, w_mat, scale_x, scale_w):
    x = x.astype(jnp.float8_e4m3fn)
    w_mat = w_mat.astype(jnp.float8_e5m2)
    m, k_per = x.shape
    _, n = w_mat.shape
    n_hops = N_DEV - 1

    def body(x_ref, w_ref, sx_ref, sw_ref, out_ref,
             rx_x, rx_w, sx_sems, rxv_sems, sw_sems, rwv_sems):
        my = lax.axis_index("i")
        left = (my - 1) % N_DEV
        right = (my + 1) % N_DEV

        barrier = pltpu.get_barrier_semaphore()
        for nbr in (left, right):
            pl.semaphore_signal(barrier, inc=1, device_id=(nbr,),
                                device_id_type=pl.DeviceIdType.MESH)
        pl.semaphore_wait(barrier, 2)

        def dotf(a, b):
            return lax.dot_general(
                a, b, (((1,), (0,)), ((), ())),
                preferred_element_type=jnp.float32)

        rdx0 = pltpu.make_async_remote_copy(
            src_ref=x_ref, dst_ref=rx_x.at[0],
            send_sem=sx_sems.at[0], recv_sem=rxv_sems.at[0],
            device_id=(right,), device_id_type=pl.DeviceIdType.MESH)
        rdw0 = pltpu.make_async_remote_copy(
            src_ref=w_ref, dst_ref=rx_w.at[0],
            send_sem=sw_sems.at[0], recv_sem=rwv_sems.at[0],
            device_id=(right,), device_id_type=pl.DeviceIdType.MESH)
        rdx0.start()
        rdw0.start()
        out_ref[...] = dotf(x_ref[...], w_ref[...])
        rdx0.wait()
        rdw0.wait()

        def hop(h, _):
            rdx = pltpu.make_async_remote_copy(
                src_ref=rx_x.at[h - 1], dst_ref=rx_x.at[h],
                send_sem=sx_sems.at[h], recv_sem=rxv_sems.at[h],
                device_id=(right,), device_id_type=pl.DeviceIdType.MESH)
            rdw = pltpu.make_async_remote_copy(
                src_ref=rx_w.at[h - 1], dst_ref=rx_w.at[h],
                send_sem=sw_sems.at[h], recv_sem=rwv_sems.at[h],
                device_id=(right,), device_id_type=pl.DeviceIdType.MESH)
            rdx.start()
            rdw.start()
            out_ref[...] += dotf(rx_x[h - 1], rx_w[h - 1])
            rdx.wait()
            rdw.wait()
            return _

        lax.fori_loop(1, n_hops, hop, 0, unroll=False)

        out_ref[...] += dotf(rx_x[n_hops - 1], rx_w[n_hops - 1])
        scale = sx_ref[0] * sw_ref[0]
        out_ref[...] = jnp.maximum(out_ref[...] * scale, 0.0)

    return pl.pallas_call(
        body,
        out_shape=jax.ShapeDtypeStruct((m, n), jnp.float32),
        in_specs=[
            pl.BlockSpec(memory_space=pltpu.VMEM),
            pl.BlockSpec(memory_space=pltpu.VMEM),
            pl.BlockSpec(memory_space=pltpu.SMEM),
            pl.BlockSpec(memory_space=pltpu.SMEM),
        ],
        out_specs=pl.BlockSpec(memory_space=pltpu.VMEM),
        scratch_shapes=[
            pltpu.VMEM((n_hops, m, k_per), x.dtype),
            pltpu.VMEM((n_hops, k_per, n), w_mat.dtype),
            pltpu.SemaphoreType.DMA((n_hops,)),
            pltpu.SemaphoreType.DMA((n_hops,)),
            pltpu.SemaphoreType.DMA((n_hops,)),
            pltpu.SemaphoreType.DMA((n_hops,)),
        ],
        compiler_params=pltpu.CompilerParams(
            collective_id=0,
            vmem_limit_bytes=100 * 1024 * 1024,
        ),
    )(x, w_mat, scale_x, scale_w)


# device time: 211865 ns/iter; 1.5616x vs baseline; 1.5616x over previous
import jax
import jax.numpy as jnp
from jax import lax
from jax.experimental import pallas as pl
from jax.experimental.pallas import tpu as pltpu

N_DEV = 16
N_R = N_DEV // 2
N_L = N_DEV - 1 - N_R


def kernel(x, w_mat, scale_x, scale_w):
    x = x.astype(jnp.float8_e4m3fn)
    w_mat = w_mat.astype(jnp.float8_e5m2)
    m, k_per = x.shape
    _, n = w_mat.shape

    def body(x_ref, w_ref, sx_ref, sw_ref, out_ref,
             rxr_x, rxr_w, rxl_x, rxl_w,
             sr_x, rr_x, sr_w, rr_w,
             sl_x, rl_x, sl_w, rl_w):
        my = lax.axis_index("i")
        left = (my - 1) % N_DEV
        right = (my + 1) % N_DEV

        barrier = pltpu.get_barrier_semaphore()
        for nbr in (left, right):
            pl.semaphore_signal(barrier, inc=1, device_id=(nbr,),
                                device_id_type=pl.DeviceIdType.MESH)
        pl.semaphore_wait(barrier, 2)

        def dotf(a, b):
            return lax.dot_general(
                a, b, (((1,), (0,)), ((), ())),
                preferred_element_type=jnp.float32)

        def mk(src, dst, ssem, rsem, dev):
            return pltpu.make_async_remote_copy(
                src_ref=src, dst_ref=dst, send_sem=ssem, recv_sem=rsem,
                device_id=(dev,), device_id_type=pl.DeviceIdType.MESH)

        rds = [mk(x_ref, rxr_x.at[0], sr_x.at[0], rr_x.at[0], right),
               mk(w_ref, rxr_w.at[0], sr_w.at[0], rr_w.at[0], right),
               mk(x_ref, rxl_x.at[0], sl_x.at[0], rl_x.at[0], left),
               mk(w_ref, rxl_w.at[0], sl_w.at[0], rl_w.at[0], left)]
        for r in rds:
            r.start()
        out_ref[...] = dotf(x_ref[...], w_ref[...])
        for r in rds:
            r.wait()

        def step(s, _):
            rds = [mk(rxr_x.at[s - 1], rxr_x.at[s], sr_x.at[s], rr_x.at[s], right),
                   mk(rxr_w.at[s - 1], rxr_w.at[s], sr_w.at[s], rr_w.at[s], right),
                   mk(rxl_x.at[s - 1], rxl_x.at[s], sl_x.at[s], rl_x.at[s], left),
                   mk(rxl_w.at[s - 1], rxl_w.at[s], sl_w.at[s], rl_w.at[s], left)]
            for r in rds:
                r.start()
            out_ref[...] += dotf(rxr_x[s - 1], rxr_w[s - 1])
            out_ref[...] += dotf(rxl_x[s - 1], rxl_w[s - 1])
            for r in rds:
                r.wait()
            return _

        lax.fori_loop(1, N_L, step, 0, unroll=False)

        s = N_L
        rdx = mk(rxr_x.at[s - 1], rxr_x.at[s], sr_x.at[s], rr_x.at[s], right)
        rdw = mk(rxr_w.at[s - 1], rxr_w.at[s], sr_w.at[s], rr_w.at[s], right)
        rdx.start()
        rdw.start()
        out_ref[...] += dotf(rxr_x[s - 1], rxr_w[s - 1])
        out_ref[...] += dotf(rxl_x[s - 1], rxl_w[s - 1])
        rdx.wait()
        rdw.wait()
        out_ref[...] += dotf(rxr_x[N_R - 1], rxr_w[N_R - 1])

        scale = sx_ref[0] * sw_ref[0]
        out_ref[...] = jnp.maximum(out_ref[...] * scale, 0.0)

    return pl.pallas_call(
        body,
        out_shape=jax.ShapeDtypeStruct((m, n), jnp.float32),
        in_specs=[
            pl.BlockSpec(memory_space=pltpu.VMEM),
            pl.BlockSpec(memory_space=pltpu.VMEM),
            pl.BlockSpec(memory_space=pltpu.SMEM),
            pl.BlockSpec(memory_space=pltpu.SMEM),
        ],
        out_specs=pl.BlockSpec(memory_space=pltpu.VMEM),
        scratch_shapes=[
            pltpu.VMEM((N_R, m, k_per), x.dtype),
            pltpu.VMEM((N_R, k_per, n), w_mat.dtype),
            pltpu.VMEM((N_L, m, k_per), x.dtype),
            pltpu.VMEM((N_L, k_per, n), w_mat.dtype),
            pltpu.SemaphoreType.DMA((N_R,)),
            pltpu.SemaphoreType.DMA((N_R,)),
            pltpu.SemaphoreType.DMA((N_R,)),
            pltpu.SemaphoreType.DMA((N_R,)),
            pltpu.SemaphoreType.DMA((N_L,)),
            pltpu.SemaphoreType.DMA((N_L,)),
            pltpu.SemaphoreType.DMA((N_L,)),
            pltpu.SemaphoreType.DMA((N_L,)),
        ],
        compiler_params=pltpu.CompilerParams(
            collective_id=0,
            vmem_limit_bytes=100 * 1024 * 1024,
        ),
    )(x, w_mat, scale_x, scale_w)


# device time: 203954 ns/iter; 1.6222x vs baseline; 1.0388x over previous
import jax
import jax.numpy as jnp
from jax import lax
from jax.experimental import pallas as pl
from jax.experimental.pallas import tpu as pltpu

N_DEV = 16
N_R = N_DEV // 2
N_L = N_DEV - 1 - N_R

_RING = [0, 1, 5, 9, 13, 14, 10, 6, 2, 3, 7, 11, 15, 12, 8, 4]
_NEXT = [0] * N_DEV
_PREV = [0] * N_DEV
for _i, _d in enumerate(_RING):
    _NEXT[_d] = _RING[(_i + 1) % N_DEV]
    _PREV[_d] = _RING[(_i - 1) % N_DEV]


def kernel(x, w_mat, scale_x, scale_w):
    x = x.astype(jnp.float8_e4m3fn)
    w_mat = w_mat.astype(jnp.float8_e5m2)
    m, k_per = x.shape
    _, n = w_mat.shape

    my_pos = lax.axis_index("i")
    nxt = jnp.asarray(_NEXT, jnp.int32)[my_pos][None]
    prv = jnp.asarray(_PREV, jnp.int32)[my_pos][None]

    def body(x_ref, w_ref, sx_ref, sw_ref, nxt_ref, prv_ref, out_ref,
             rxr_x, rxr_w, rxl_x, rxl_w,
             sr_x, rr_x, sr_w, rr_w,
             sl_x, rl_x, sl_w, rl_w):
        right = nxt_ref[0]
        left = prv_ref[0]

        barrier = pltpu.get_barrier_semaphore()
        for nbr in (left, right):
            pl.semaphore_signal(barrier, inc=1, device_id=(nbr,),
                                device_id_type=pl.DeviceIdType.MESH)
        pl.semaphore_wait(barrier, 2)

        def dotf(a, b):
            return lax.dot_general(
                a, b, (((1,), (0,)), ((), ())),
                preferred_element_type=jnp.float32)

        def mk(src, dst, ssem, rsem, dev):
            return pltpu.make_async_remote_copy(
                src_ref=src, dst_ref=dst, send_sem=ssem, recv_sem=rsem,
                device_id=(dev,), device_id_type=pl.DeviceIdType.MESH)

        rds = [mk(x_ref, rxr_x.at[0], sr_x.at[0], rr_x.at[0], right),
               mk(w_ref, rxr_w.at[0], sr_w.at[0], rr_w.at[0], right),
               mk(x_ref, rxl_x.at[0], sl_x.at[0], rl_x.at[0], left),
               mk(w_ref, rxl_w.at[0], sl_w.at[0], rl_w.at[0], left)]
        for r in rds:
            r.start()
        out_ref[...] = dotf(x_ref[...], w_ref[...])
        for r in rds:
            r.wait()

        def step(s, _):
            rds = [mk(rxr_x.at[s - 1], rxr_x.at[s], sr_x.at[s], rr_x.at[s], right),
                   mk(rxr_w.at[s - 1], rxr_w.at[s], sr_w.at[s], rr_w.at[s], right),
                   mk(rxl_x.at[s - 1], rxl_x.at[s], sl_x.at[s], rl_x.at[s], left),
                   mk(rxl_w.at[s - 1], rxl_w.at[s], sl_w.at[s], rl_w.at[s], left)]
            for r in rds:
                r.start()
            out_ref[...] += dotf(rxr_x[s - 1], rxr_w[s - 1])
            out_ref[...] += dotf(rxl_x[s - 1], rxl_w[s - 1])
            for r in rds:
                r.wait()
            return _

        lax.fori_loop(1, N_L, step, 0, unroll=False)

        s = N_L
        rdx = mk(rxr_x.at[s - 1], rxr_x.at[s], sr_x.at[s], rr_x.at[s], right)
        rdw = mk(rxr_w.at[s - 1], rxr_w.at[s], sr_w.at[s], rr_w.at[s], right)
        rdx.start()
        rdw.start()
        out_ref[...] += dotf(rxr_x[s - 1], rxr_w[s - 1])
        out_ref[...] += dotf(rxl_x[s - 1], rxl_w[s - 1])
        rdx.wait()
        rdw.wait()
        out_ref[...] += dotf(rxr_x[N_R - 1], rxr_w[N_R - 1])

        scale = sx_ref[0] * sw_ref[0]
        out_ref[...] = jnp.maximum(out_ref[...] * scale, 0.0)

    return pl.pallas_call(
        body,
        out_shape=jax.ShapeDtypeStruct((m, n), jnp.float32),
        in_specs=[
            pl.BlockSpec(memory_space=pltpu.VMEM),
            pl.BlockSpec(memory_space=pltpu.VMEM),
            pl.BlockSpec(memory_space=pltpu.SMEM),
            pl.BlockSpec(memory_space=pltpu.SMEM),
            pl.BlockSpec(memory_space=pltpu.SMEM),
            pl.BlockSpec(memory_space=pltpu.SMEM),
        ],
        out_specs=pl.BlockSpec(memory_space=pltpu.VMEM),
        scratch_shapes=[
            pltpu.VMEM((N_R, m, k_per), x.dtype),
            pltpu.VMEM((N_R, k_per, n), w_mat.dtype),
            pltpu.VMEM((N_L, m, k_per), x.dtype),
            pltpu.VMEM((N_L, k_per, n), w_mat.dtype),
            pltpu.SemaphoreType.DMA((N_R,)),
            pltpu.SemaphoreType.DMA((N_R,)),
            pltpu.SemaphoreType.DMA((N_R,)),
            pltpu.SemaphoreType.DMA((N_R,)),
            pltpu.SemaphoreType.DMA((N_L,)),
            pltpu.SemaphoreType.DMA((N_L,)),
            pltpu.SemaphoreType.DMA((N_L,)),
            pltpu.SemaphoreType.DMA((N_L,)),
        ],
        compiler_params=pltpu.CompilerParams(
            collective_id=0,
            vmem_limit_bytes=100 * 1024 * 1024,
        ),
    )(x, w_mat, scale_x, scale_w, nxt, prv)


# device time: 203382 ns/iter; 1.6267x vs baseline; 1.0028x over previous
import jax
import jax.numpy as jnp
from jax import lax
from jax.experimental import pallas as pl
from jax.experimental.pallas import tpu as pltpu

N_DEV = 16
N_R = N_DEV // 2
N_L = N_DEV - 1 - N_R

B = 3

_RING = [0, 1, 5, 9, 13, 14, 10, 6, 2, 3, 7, 11, 15, 12, 8, 4]
_NEXT = [0] * N_DEV
_PREV = [0] * N_DEV
for _i, _d in enumerate(_RING):
    _NEXT[_d] = _RING[(_i + 1) % N_DEV]
    _PREV[_d] = _RING[(_i - 1) % N_DEV]


def kernel(x, w_mat, scale_x, scale_w):
    x = x.astype(jnp.float8_e4m3fn)
    w_mat = w_mat.astype(jnp.float8_e5m2)
    m, k_per = x.shape
    _, n = w_mat.shape

    my_pos = lax.axis_index("i")
    nxt = jnp.asarray(_NEXT, jnp.int32)[my_pos][None]
    prv = jnp.asarray(_PREV, jnp.int32)[my_pos][None]

    def body(x_ref, w_ref, sx_ref, sw_ref, nxt_ref, prv_ref, out_ref,
             rx_x, rx_w,
             sr_x, rr_x, sr_w, rr_w,
             sl_x, rl_x, sl_w, rl_w):
        right = nxt_ref[0]
        left = prv_ref[0]

        barrier = pltpu.get_barrier_semaphore()
        for nbr in (left, right):
            pl.semaphore_signal(barrier, inc=1, device_id=(nbr,),
                                device_id_type=pl.DeviceIdType.MESH)
        pl.semaphore_wait(barrier, 2)

        def dotf(a, b):
            return lax.dot_general(
                a, b, (((1,), (0,)), ((), ())),
                preferred_element_type=jnp.float32)

        def mk(src, dst, ssem, rsem, dev):
            return pltpu.make_async_remote_copy(
                src_ref=src, dst_ref=dst, send_sem=ssem, recv_sem=rsem,
                device_id=(dev,), device_id_type=pl.DeviceIdType.MESH)

        def mk4(sxr, dxr, sxl, dxl, u):
            return [mk(sxr, rx_x.at[u, :, 0:k_per], sr_x.at[u], rr_x.at[u], right),
                    mk(dxr, rx_w.at[u, 0:k_per, :], sr_w.at[u], rr_w.at[u], right),
                    mk(sxl, rx_x.at[u, :, k_per:2 * k_per], sl_x.at[u], rl_x.at[u], left),
                    mk(dxl, rx_w.at[u, k_per:2 * k_per, :], sl_w.at[u], rl_w.at[u], left)]

        rds = mk4(x_ref, w_ref, x_ref, w_ref, 0)
        for r in rds:
            r.start()
        out_ref[...] = dotf(x_ref[...], w_ref[...])
        for r in rds:
            r.wait()

        def step(s, _):
            u = s % B
            p = (s - 1) % B
            rds = mk4(rx_x.at[p, :, 0:k_per],
                      rx_w.at[p, 0:k_per, :],
                      rx_x.at[p, :, k_per:2 * k_per],
                      rx_w.at[p, k_per:2 * k_per, :], u)
            for r in rds:
                r.start()
            out_ref[...] += dotf(rx_x[p], rx_w[p])
            for r in rds:
                r.wait()
            return _

        lax.fori_loop(1, N_L, step, 0, unroll=False)

        u = N_L % B
        p = (N_L - 1) % B
        rdx = mk(rx_x.at[p, :, 0:k_per], rx_x.at[u, :, 0:k_per],
                 sr_x.at[u], rr_x.at[u], right)
        rdw = mk(rx_w.at[p, 0:k_per, :], rx_w.at[u, 0:k_per, :],
                 sr_w.at[u], rr_w.at[u], right)
        rdx.start()
        rdw.start()
        out_ref[...] += dotf(rx_x[p], rx_w[p])
        rdx.wait()
        rdw.wait()

        scale = sx_ref[0] * sw_ref[0]
        out_ref[...] = jnp.maximum(
            (out_ref[...]
             + dotf(rx_x[u, :, 0:k_per], rx_w[u, 0:k_per, :]))
            * scale,
            0.0)

    return pl.pallas_call(
        body,
        out_shape=jax.ShapeDtypeStruct((m, n), jnp.float32),
        in_specs=[
            pl.BlockSpec(memory_space=pltpu.VMEM),
            pl.BlockSpec(memory_space=pltpu.VMEM),
            pl.BlockSpec(memory_space=pltpu.SMEM),
            pl.BlockSpec(memory_space=pltpu.SMEM),
            pl.BlockSpec(memory_space=pltpu.SMEM),
            pl.BlockSpec(memory_space=pltpu.SMEM),
        ],
        out_specs=pl.BlockSpec(memory_space=pltpu.VMEM),
        scratch_shapes=[
            pltpu.VMEM((B, m, 2 * k_per), x.dtype),
            pltpu.VMEM((B, 2 * k_per, n), w_mat.dtype),
            pltpu.SemaphoreType.DMA((B,)),
            pltpu.SemaphoreType.DMA((B,)),
            pltpu.SemaphoreType.DMA((B,)),
            pltpu.SemaphoreType.DMA((B,)),
            pltpu.SemaphoreType.DMA((B,)),
            pltpu.SemaphoreType.DMA((B,)),
            pltpu.SemaphoreType.DMA((B,)),
            pltpu.SemaphoreType.DMA((B,)),
        ],
        compiler_params=pltpu.CompilerParams(
            collective_id=0,
            vmem_limit_bytes=100 * 1024 * 1024,
        ),
    )(x, w_mat, scale_x, scale_w, nxt, prv)


# device time: 194979 ns/iter; 1.6968x vs baseline; 1.0431x over previous
import jax
import jax.numpy as jnp
from jax import lax
from jax.experimental import pallas as pl
from jax.experimental.pallas import tpu as pltpu

N_DEV = 16
N_R = N_DEV // 2
N_L = N_DEV - 1 - N_R

B = 3

_RING = [0, 1, 5, 9, 13, 14, 10, 6, 2, 3, 7, 11, 15, 12, 8, 4]
_NEXT = [0] * N_DEV
_PREV = [0] * N_DEV
for _i, _d in enumerate(_RING):
    _NEXT[_d] = _RING[(_i + 1) % N_DEV]
    _PREV[_d] = _RING[(_i - 1) % N_DEV]


def kernel(x, w_mat, scale_x, scale_w):
    m, k_per = x.shape
    _, n = w_mat.shape

    def body(x_ref, w_ref, sx_ref, sw_ref, out_ref,
             rx_x, rx_w,
             sr_x, rr_x, sr_w, rr_w,
             sl_x, rl_x, sl_w, rl_w):
        my = lax.axis_index("i")
        right = jnp.int32(_NEXT[0])
        left = jnp.int32(_PREV[0])
        for d in range(1, N_DEV):
            right = jnp.where(my == d, _NEXT[d], right)
            left = jnp.where(my == d, _PREV[d], left)

        barrier = pltpu.get_barrier_semaphore()
        for nbr in (left, right):
            pl.semaphore_signal(barrier, inc=1, device_id=(nbr,),
                                device_id_type=pl.DeviceIdType.MESH)
        pl.semaphore_wait(barrier, 2)

        def dotf(a, b):
            return lax.dot_general(
                a, b, (((1,), (0,)), ((), ())),
                preferred_element_type=jnp.float32)

        def mk(src, dst, ssem, rsem, dev):
            return pltpu.make_async_remote_copy(
                src_ref=src, dst_ref=dst, send_sem=ssem, recv_sem=rsem,
                device_id=(dev,), device_id_type=pl.DeviceIdType.MESH)

        def mk4(sxr, dxr, sxl, dxl, u):
            return [mk(sxr, rx_x.at[u, :, 0:k_per], sr_x.at[u], rr_x.at[u], right),
                    mk(dxr, rx_w.at[u, 0:k_per, :], sr_w.at[u], rr_w.at[u], right),
                    mk(sxl, rx_x.at[u, :, k_per:2 * k_per], sl_x.at[u], rl_x.at[u], left),
                    mk(dxl, rx_w.at[u, k_per:2 * k_per, :], sl_w.at[u], rl_w.at[u], left)]

        own_x = rx_x.at[2, :, 0:k_per]
        own_w = rx_w.at[2, 0:k_per, :]
        own_x[...] = x_ref[...].astype(jnp.float8_e4m3fn)
        own_w[...] = w_ref[...].astype(jnp.float8_e5m2)

        rds = mk4(own_x, own_w, own_x, own_w, 0)
        for r in rds:
            r.start()
        out_ref[...] = dotf(own_x[...], own_w[...])
        for r in rds:
            r.wait()

        def step(s, _):
            u = s % B
            p = (s - 1) % B
            rds = mk4(rx_x.at[p, :, 0:k_per],
                      rx_w.at[p, 0:k_per, :],
                      rx_x.at[p, :, k_per:2 * k_per],
                      rx_w.at[p, k_per:2 * k_per, :], u)
            for r in rds:
                r.start()
            out_ref[...] += dotf(rx_x[p], rx_w[p])
            for r in rds:
                r.wait()
            return _

        lax.fori_loop(1, N_L, step, 0, unroll=False)

        u = N_L % B
        p = (N_L - 1) % B
        rdx = mk(rx_x.at[p, :, 0:k_per], rx_x.at[u, :, 0:k_per],
                 sr_x.at[u], rr_x.at[u], right)
        rdw = mk(rx_w.at[p, 0:k_per, :], rx_w.at[u, 0:k_per, :],
                 sr_w.at[u], rr_w.at[u], right)
        rdx.start()
        rdw.start()
        out_ref[...] += dotf(rx_x[p], rx_w[p])
        rdx.wait()
        rdw.wait()

        scale = sx_ref[0] * sw_ref[0]
        out_ref[...] = jnp.maximum(
            (out_ref[...]
             + dotf(rx_x[u, :, 0:k_per], rx_w[u, 0:k_per, :]))
            * scale,
            0.0)

    return pl.pallas_call(
        body,
        out_shape=jax.ShapeDtypeStruct((m, n), jnp.float32),
        in_specs=[
            pl.BlockSpec(memory_space=pltpu.VMEM),
            pl.BlockSpec(memory_space=pltpu.VMEM),
            pl.BlockSpec(memory_space=pltpu.SMEM),
            pl.BlockSpec(memory_space=pltpu.SMEM),
        ],
        out_specs=pl.BlockSpec(memory_space=pltpu.VMEM),
        scratch_shapes=[
            pltpu.VMEM((B, m, 2 * k_per), jnp.float8_e4m3fn),
            pltpu.VMEM((B, 2 * k_per, n), jnp.float8_e5m2),
            pltpu.SemaphoreType.DMA((B,)),
            pltpu.SemaphoreType.DMA((B,)),
            pltpu.SemaphoreType.DMA((B,)),
            pltpu.SemaphoreType.DMA((B,)),
            pltpu.SemaphoreType.DMA((B,)),
            pltpu.SemaphoreType.DMA((B,)),
            pltpu.SemaphoreType.DMA((B,)),
            pltpu.SemaphoreType.DMA((B,)),
        ],
        compiler_params=pltpu.CompilerParams(
            collective_id=0,
            vmem_limit_bytes=100 * 1024 * 1024,
        ),
    )(x, w_mat, scale_x, scale_w)


# device time: 185252 ns/iter; 1.7859x vs baseline; 1.0525x over previous
import jax
import jax.numpy as jnp
from jax import lax
from jax.experimental import pallas as pl
from jax.experimental.pallas import tpu as pltpu

N_DEV = 16
N_R = N_DEV // 2
N_L = N_DEV - 1 - N_R

B = 3

_RING = [0, 1, 5, 9, 13, 14, 10, 6, 2, 3, 7, 11, 15, 12, 8, 4]
_NEXT = [0] * N_DEV
_PREV = [0] * N_DEV
for _i, _d in enumerate(_RING):
    _NEXT[_d] = _RING[(_i + 1) % N_DEV]
    _PREV[_d] = _RING[(_i - 1) % N_DEV]


def kernel(x, w_mat, scale_x, scale_w):
    m, k_per = x.shape
    _, n = w_mat.shape
    m2 = m // 2
    k2 = k_per // 2

    def body(x_ref, w_ref, sx_ref, sw_ref, out_ref,
             rx_x, rx_w,
             sr_x, rr_x, sr_w, rr_w,
             sl_x, rl_x, sl_w, rl_w):
        my = lax.axis_index("i")
        right = jnp.int32(_NEXT[0])
        left = jnp.int32(_PREV[0])
        for d in range(1, N_DEV):
            right = jnp.where(my == d, _NEXT[d], right)
            left = jnp.where(my == d, _PREV[d], left)

        barrier = pltpu.get_barrier_semaphore()
        for nbr in (left, right):
            pl.semaphore_signal(barrier, inc=1, device_id=(nbr,),
                                device_id_type=pl.DeviceIdType.MESH)
        pl.semaphore_wait(barrier, 2)

        def dotf(a, b):
            return lax.dot_general(
                a, b, (((1,), (0,)), ((), ())),
                preferred_element_type=jnp.float32)

        def mk(src, dst, ssem, rsem, dev):
            return pltpu.make_async_remote_copy(
                src_ref=src, dst_ref=dst, send_sem=ssem, recv_sem=rsem,
                device_id=(dev,), device_id_type=pl.DeviceIdType.MESH)

        def x_half(u, side, h):
            lo = 0 if side == 0 else k_per
            return rx_x.at[u, (h * m2):((h + 1) * m2), lo:lo + k_per]

        def w_half(u, side, h):
            lo = (0 if side == 0 else k_per) + h * k2
            return rx_w.at[u, lo:lo + k2, :]

        def flows(u, p, srcs):
            sxr, swr, sxl, swl = srcs
            return [
                (sxr[0], x_half(u, 0, 0), sr_x.at[u, 0], rr_x.at[u, 0], right),
                (sxr[1], x_half(u, 0, 1), sr_x.at[u, 1], rr_x.at[u, 1], right),
                (swr[0], w_half(u, 0, 0), sr_w.at[u, 0], rr_w.at[u, 0], right),
                (swr[1], w_half(u, 0, 1), sr_w.at[u, 1], rr_w.at[u, 1], right),
                (sxl[0], x_half(u, 1, 0), sl_x.at[u, 0], rl_x.at[u, 0], left),
                (sxl[1], x_half(u, 1, 1), sl_x.at[u, 1], rl_x.at[u, 1], left),
                (swl[0], w_half(u, 1, 0), sl_w.at[u, 0], rl_w.at[u, 0], left),
                (swl[1], w_half(u, 1, 1), sl_w.at[u, 1], rl_w.at[u, 1], left),
            ]

        def slot_srcs(p, side):
            return ([x_half(p, side, 0), x_half(p, side, 1)],
                    [w_half(p, side, 0), w_half(p, side, 1)])

        own_x = rx_x.at[2, :, 0:k_per]
        own_w = rx_w.at[2, 0:k_per, :]
        own_x[...] = x_ref[...].astype(jnp.float8_e4m3fn)
        own_w[...] = w_ref[...].astype(jnp.float8_e5m2)
        oxs, ows = slot_srcs(2, 0)

        rds0 = [mk(*f) for f in flows(0, -1, (oxs, ows, oxs, ows))]
        for r in rds0:
            r.start()
        out_ref[...] = dotf(own_x[...], own_w[...])
        for r in rds0:
            r.wait_send()

        def step(s, _):
            u = s % B
            p = (s - 1) % B
            xs_r, ws_r = slot_srcs(p, 0)
            xs_l, ws_l = slot_srcs(p, 1)
            prev = flows(p, -1, (xs_r, ws_r, xs_l, ws_l))
            rds = [mk(*f) for f in flows(u, p, (xs_r, ws_r, xs_l, ws_l))]
            for a, b in ((0, 4), (1, 5), (2, 6), (3, 7)):
                mk(*prev[a]).wait_recv()
                mk(*prev[b]).wait_recv()
                rds[a].start()
                rds[b].start()
            out_ref[...] += dotf(rx_x[p], rx_w[p])
            for r in rds:
                r.wait_send()
            return _

        lax.fori_loop(1, N_L, step, 0, unroll=False)

        u = N_L % B
        p = (N_L - 1) % B
        xs_r, ws_r = slot_srcs(p, 0)
        xs_l, ws_l = slot_srcs(p, 1)
        prev = flows(p, -1, (xs_r, ws_r, xs_l, ws_l))
        rds = [mk(*f) for f in flows(u, p, (xs_r, ws_r, xs_l, ws_l))[0:4]]
        for i in (0, 1, 2, 3):
            mk(*prev[i]).wait_recv()
            mk(*prev[i + 4]).wait_recv()
            rds[i].start()
        out_ref[...] += dotf(rx_x[p], rx_w[p])
        for r in rds:
            r.wait_send()

        last = flows(u, p, (xs_r, ws_r, xs_l, ws_l))
        for i in (0, 1, 2, 3):
            mk(*last[i]).wait_recv()

        scale = sx_ref[0] * sw_ref[0]
        out_ref[...] = jnp.maximum(
            (out_ref[...]
             + dotf(rx_x[u, :, 0:k_per], rx_w[u, 0:k_per, :]))
            * scale,
            0.0)

    return pl.pallas_call(
        body,
        out_shape=jax.ShapeDtypeStruct((m, n), jnp.float32),
        in_specs=[
            pl.BlockSpec(memory_space=pltpu.VMEM),
            pl.BlockSpec(memory_space=pltpu.VMEM),
            pl.BlockSpec(memory_space=pltpu.SMEM),
            pl.BlockSpec(memory_space=pltpu.SMEM),
        ],
        out_specs=pl.BlockSpec(memory_space=pltpu.VMEM),
        scratch_shapes=[
            pltpu.VMEM((B, m, 2 * k_per), jnp.float8_e4m3fn),
            pltpu.VMEM((B, 2 * k_per, n), jnp.float8_e5m2),
            pltpu.SemaphoreType.DMA((B, 2)),
            pltpu.SemaphoreType.DMA((B, 2)),
            pltpu.SemaphoreType.DMA((B, 2)),
            pltpu.SemaphoreType.DMA((B, 2)),
            pltpu.SemaphoreType.DMA((B, 2)),
            pltpu.SemaphoreType.DMA((B, 2)),
            pltpu.SemaphoreType.DMA((B, 2)),
            pltpu.SemaphoreType.DMA((B, 2)),
        ],
        compiler_params=pltpu.CompilerParams(
            collective_id=0,
            vmem_limit_bytes=100 * 1024 * 1024,
        ),
    )(x, w_mat, scale_x, scale_w)
